# baseline (speedup 1.0000x reference)
import jax
import jax.numpy as jnp
from jax.experimental import pallas as pl
from jax.experimental.pallas import tpu as pltpu

N = 4096
F32 = jnp.float32
BF16 = jnp.bfloat16
BLK = 128
NB = N // BLK
BLK_E = 256
NB_E = N // BLK_E
BLK_S = 512


def _dot(a, b):
    return jnp.dot(a.astype(BF16), b.astype(BF16),
                   preferred_element_type=F32)


def _xw_kernel(x_ref, w_ref, o_ref):
    o_ref[...] = _dot(x_ref[...], w_ref[...]).astype(BF16)


def _km_kernel(f_ref, s_ref, xw_ref, bab_ref, w23_ref, wg2a_ref, wg2b_ref,
               bc_ref, bg_ref,
               smu_ref, slv_ref, fmu_ref, flv_ref, h1_ref, h1t_ref,
               h2_ref, a2_ref,
               cache_bf, r1s, r2s, h2ts, h2s_bf):
    g = pl.program_id(0)
    i = pl.program_id(1)
    rows = pl.ds(i * BLK, BLK)

    @pl.when(g == 0)
    def _phase0():
        fb = f_ref[...].astype(BF16)
        cache_bf[rows, :] = fb
        xw = xw_ref[...]
        b = bab_ref[...]
        fh1 = jax.nn.relu(_dot(fb, xw[:, :256]) + b[:, :256]).astype(BF16)
        t2 = jax.nn.relu(_dot(fb, xw[:, 512:768]) + b[:, 512:768]).astype(BF16)
        r1s[rows, 256:512] = _dot(fh1, w23_ref[...]).astype(BF16)
        r2s[rows, :] = _dot(t2, wg2b_ref[...]).astype(BF16)

    @pl.when(g == 1)
    def _phase1():
        fb = cache_bf[rows, :]
        h2b = jax.nn.relu(_dot(fb, r2s[...]) + bg_ref[...])
        h2_ref[...] = h2b
        h2ts[:, rows] = h2b.T.astype(BF16)
        h2s_bf[rows, :] = h2b.astype(BF16)
        sb = s_ref[...].astype(BF16)
        cache_bf[rows, :] = sb
        xw = xw_ref[...]
        b = bab_ref[...]
        pa = _dot(sb, xw[:, :512])
        sh1 = jax.nn.relu(pa[:, :256] + b[:, :256]).astype(BF16)
        t1 = jax.nn.relu(pa[:, 256:512] + b[:, 256:512]).astype(BF16)
        r1s[rows, 0:256] = _dot(sh1, w23_ref[...]).astype(BF16)
        r1s[rows, 512:640] = _dot(t1, wg2a_ref[...]).astype(BF16)

    @pl.when(g == 2)
    def _phase2():
        sb = cache_bf[rows, :]
        p = jax.nn.relu(_dot(sb, r1s[...]) + bc_ref[...])
        smu_ref[...] = p[:, 0:128]
        slv_ref[...] = p[:, 128:256]
        fmu_ref[...] = p[:, 256:384]
        flv_ref[...] = p[:, 384:512]
        h1b = p[:, 512:640]
        h1_ref[...] = h1b
        h1t_ref[...] = h1b.T.astype(BF16)
        a2_ref[...] = jax.nn.sigmoid(_dot(h2s_bf[rows, :], h2ts[...]))


def _ke_kernel(h1_ref, h2_ref, h1t_ref, m1_ref, m2_ref, bm2_ref,
               m3_ref, bm3_ref, ls_ref,
               a1_ref, e1_ref, e2_ref, out_ref, els_ref):
    r1 = h1_ref[...]
    r2 = h2_ref[...]
    a1_ref[...] = jax.nn.sigmoid(_dot(r1, h1t_ref[...]))
    n1 = jnp.sqrt(jnp.sum(r1 * r1, axis=1, keepdims=True))
    e1_ref[...] = r1 / n1
    n2 = jnp.sqrt(jnp.sum(r2 * r2, axis=1, keepdims=True))
    e2_ref[...] = r2 / n2
    z = jnp.concatenate([r1, r2], axis=1)
    t = _dot(z, m1_ref[...])
    t = _dot(t, m2_ref[...]) + bm2_ref[...]
    t = _dot(t, m3_ref[...]) + bm3_ref[...]
    m = jnp.max(t, axis=1, keepdims=True)
    out_ref[...] = t - m - jnp.log(jnp.sum(jnp.exp(t - m), axis=1,
                                           keepdims=True))
    els_ref[...] = jnp.exp(ls_ref[...])


def kernel(x, sadj, fadj, W1, b1, W2, b2, W3, b3, Wg1a, bg1a, Wg2a, bg2a,
           Wg1b, bg1b, Wg2b, bg2b, M1, M2, bM2, M3, bM3, logit_scale):
    Wc = jnp.concatenate([W1, Wg1a, Wg1b], axis=1)
    XW = pl.pallas_call(
        _xw_kernel,
        grid=(N // BLK_S,),
        in_specs=[
            pl.BlockSpec((BLK_S, 512), lambda i: (i, 0)),
            pl.BlockSpec((512, 768), lambda i: (0, 0)),
        ],
        out_specs=pl.BlockSpec((BLK_S, 768), lambda i: (i, 0)),
        out_shape=jax.ShapeDtypeStruct((N, 768), BF16),
        compiler_params=pltpu.CompilerParams(
            dimension_semantics=("parallel",)),
    )(x, Wc)

    last = NB - 1
    f_spec = pl.BlockSpec((BLK, N),
                          lambda g, i: (jnp.where(g == 0, i, last), 0))
    s_spec = pl.BlockSpec((BLK, N),
                          lambda g, i: (jnp.where(g == 1, i,
                                                  jnp.where(g == 2, last, 0)),
                                        0))
    res2 = lambda shape: pl.BlockSpec(shape, lambda g, i: (0, 0))
    p1b = lambda w: pl.BlockSpec((BLK, w),
                                 lambda g, i: (jnp.where(g == 1, i,
                                                         jnp.where(g == 2,
                                                                   last, 0)),
                                               0))
    p2b = lambda w: pl.BlockSpec((BLK, w),
                                 lambda g, i: (jnp.where(g == 2, i, 0), 0))
    p2t = pl.BlockSpec((128, BLK),
                       lambda g, i: (0, jnp.where(g == 2, i, 0)))

    bab = jnp.concatenate([b1, bg1a, bg1b]).reshape(1, 768)
    w23 = jnp.concatenate([W2, W3], axis=1).astype(BF16)
    bc = jnp.concatenate([b2, b3, b2, b3, bg2a]).reshape(1, 640)

    smu, slv, fmu, flv, h1, h1t, h2, A2 = pl.pallas_call(
        _km_kernel,
        grid=(3, NB),
        in_specs=[f_spec, s_spec, res2((N, 768)), res2((1, 768)),
                  res2((256, 256)), res2((256, 128)), res2((256, 128)),
                  res2((1, 640)), res2((1, 128))],
        out_specs=[p2b(128), p2b(128), p2b(128), p2b(128), p2b(128), p2t,
                   p1b(128), p2b(N)],
        out_shape=[jax.ShapeDtypeStruct((N, 128), F32)] * 5 +
                  [jax.ShapeDtypeStruct((128, N), BF16),
                   jax.ShapeDtypeStruct((N, 128), F32),
                   jax.ShapeDtypeStruct((N, N), F32)],
        scratch_shapes=[pltpu.VMEM((N, N), BF16),
                        pltpu.VMEM((N, 640), BF16),
                        pltpu.VMEM((N, 128), BF16),
                        pltpu.VMEM((128, N), BF16),
                        pltpu.VMEM((N, 128), BF16)],
        compiler_params=pltpu.CompilerParams(
            dimension_semantics=("arbitrary", "arbitrary"),
            vmem_limit_bytes=100 * 1024 * 1024),
    )(fadj, sadj, XW, bab, w23, Wg2a.astype(BF16), Wg2b.astype(BF16),
      bc, bg2b.reshape(1, 128))

    he = pl.BlockSpec((BLK_E, 128), lambda i: (i, 0))
    res = lambda shape: pl.BlockSpec(shape, lambda i: (0, 0))
    A1, emb1, emb2, out, els = pl.pallas_call(
        _ke_kernel,
        grid=(NB_E,),
        in_specs=[he, he, res((128, N)), res((256, 256)),
                  res((256, 128)), res((1, 128)), res((128, 16)),
                  res((1, 16)), res((1, 1))],
        out_specs=[pl.BlockSpec((BLK_E, N), lambda i: (i, 0)),
                   he, he,
                   pl.BlockSpec((BLK_E, 16), lambda i: (i, 0)),
                   pl.BlockSpec((1, 1), lambda i: (0, 0))],
        out_shape=[jax.ShapeDtypeStruct((N, N), F32),
                   jax.ShapeDtypeStruct((N, 128), F32),
                   jax.ShapeDtypeStruct((N, 128), F32),
                   jax.ShapeDtypeStruct((N, 16), F32),
                   jax.ShapeDtypeStruct((1, 1), F32)],
        compiler_params=pltpu.CompilerParams(
            dimension_semantics=("parallel",)),
    )(h1, h2, h1t, M1, M2, bM2.reshape(1, 128), M3,
      bM3.reshape(1, 16), logit_scale.reshape(1, 1))

    return (out, A1, A2, emb1, emb2, els.reshape(()), smu, slv, fmu, flv)


# speedup vs baseline: 1.1127x; 1.1127x over previous
import jax
import jax.numpy as jnp
from jax.experimental import pallas as pl
from jax.experimental.pallas import tpu as pltpu

N = 4096
F32 = jnp.float32
BF16 = jnp.bfloat16
BLK = 512
NB = N // BLK
BLKK = 256
NBK = N // BLKK
BLK_S = 512


def _dot(a, b):
    return jnp.dot(a.astype(BF16), b.astype(BF16),
                   preferred_element_type=F32)


def _xw_kernel(x_ref, w_ref, os_ref, of_ref):
    xw = _dot(x_ref[...], w_ref[...]).astype(BF16)
    os_ref[...] = xw[:, :512]
    of_ref[:, :256] = xw[:, :256]
    of_ref[:, 256:512] = xw[:, 512:768]


def _kf_kernel(f_ref, xwf_ref, babf_ref, wg2b_ref, bg_ref,
               fh1_ref, h2_ref, h2t_ref,
               fadj_bf, r2s):
    g = pl.program_id(0)
    i = pl.program_id(1)
    rows = pl.ds(i * BLK, BLK)

    @pl.when(g == 0)
    def _phase0():
        fb = f_ref[...].astype(BF16)
        fadj_bf[rows, :] = fb
        xwf = xwf_ref[...]
        b = babf_ref[...]
        fh1 = jax.nn.relu(_dot(fb, xwf[:, :256]) + b[:, :256])
        t2 = jax.nn.relu(_dot(fb, xwf[:, 256:512]) + b[:, 256:512])
        fh1_ref[...] = fh1.astype(BF16)
        r2s[rows, :] = _dot(t2.astype(BF16), wg2b_ref[...]).astype(BF16)

    @pl.when(g == 1)
    def _phase1():
        fb = fadj_bf[rows, :]
        h2b = jax.nn.relu(_dot(fb, r2s[...]) + bg_ref[...])
        h2_ref[...] = h2b
        h2t_ref[...] = h2b.T.astype(BF16)


def _ks_kernel(s_ref, xws_ref, babs_ref, fh1_ref, w23_ref, wg2a_ref, bc_ref,
               smu_ref, slv_ref, fmu_ref, flv_ref, h1_ref, h1t_ref,
               sadj_bf, r1s):
    g = pl.program_id(0)
    i = pl.program_id(1)
    rows = pl.ds(i * BLKK, BLKK)

    @pl.when(g == 0)
    def _phase0():
        sb = s_ref[...].astype(BF16)
        sadj_bf[rows, :] = sb
        xws = xws_ref[...]
        b = babs_ref[...]
        pa = _dot(sb, xws)
        sh1 = jax.nn.relu(pa[:, :256] + b[:, :256]).astype(BF16)
        t1 = jax.nn.relu(pa[:, 256:512] + b[:, 256:512]).astype(BF16)
        w23 = w23_ref[...]
        r1s[rows, 0:256] = _dot(sh1, w23).astype(BF16)
        r1s[rows, 256:512] = _dot(fh1_ref[rows, :], w23).astype(BF16)
        r1s[rows, 512:640] = _dot(t1, wg2a_ref[...]).astype(BF16)

    @pl.when(g == 1)
    def _phase1():
        sb = sadj_bf[rows, :]
        p = jax.nn.relu(_dot(sb, r1s[...]) + bc_ref[...])
        smu_ref[...] = p[:, 0:128]
        slv_ref[...] = p[:, 128:256]
        fmu_ref[...] = p[:, 256:384]
        flv_ref[...] = p[:, 384:512]
        h1b = p[:, 512:640]
        h1_ref[...] = h1b
        h1t_ref[...] = h1b.T.astype(BF16)


def _ke_kernel(h1_ref, h2_ref, h1t_ref, h2t_ref, m1_ref, m2_ref, bm2_ref,
               m3_ref, bm3_ref, ls_ref,
               a1_ref, a2_ref, e1_ref, e2_ref, out_ref, els_ref):
    r1 = h1_ref[...]
    r2 = h2_ref[...]
    a1_ref[...] = jax.nn.sigmoid(_dot(r1, h1t_ref[...]))
    a2_ref[...] = jax.nn.sigmoid(_dot(r2, h2t_ref[...]))
    n1 = jnp.sqrt(jnp.sum(r1 * r1, axis=1, keepdims=True))
    n2 = jnp.sqrt(jnp.sum(r2 * r2, axis=1, keepdims=True))
    e1_ref[...] = r1 / n1
    e2_ref[...] = r2 / n2
    z = jnp.concatenate([r1, r2], axis=1)
    t = _dot(z, m1_ref[...])
    t = _dot(t, m2_ref[...]) + bm2_ref[...]
    t = _dot(t, m3_ref[...]) + bm3_ref[...]
    m = jnp.max(t, axis=1, keepdims=True)
    out_ref[...] = t - m - jnp.log(jnp.sum(jnp.exp(t - m), axis=1,
                                           keepdims=True))
    els_ref[...] = jnp.exp(ls_ref[...])


def kernel(x, sadj, fadj, W1, b1, W2, b2, W3, b3, Wg1a, bg1a, Wg2a, bg2a,
           Wg1b, bg1b, Wg2b, bg2b, M1, M2, bM2, M3, bM3, logit_scale):
    Wc = jnp.concatenate([W1, Wg1a, Wg1b], axis=1)
    XWs, XWf = pl.pallas_call(
        _xw_kernel,
        grid=(N // BLK_S,),
        in_specs=[
            pl.BlockSpec((BLK_S, 512), lambda i: (i, 0)),
            pl.BlockSpec((512, 768), lambda i: (0, 0)),
        ],
        out_specs=[pl.BlockSpec((BLK_S, 512), lambda i: (i, 0)),
                   pl.BlockSpec((BLK_S, 512), lambda i: (i, 0))],
        out_shape=[jax.ShapeDtypeStruct((N, 512), BF16),
                   jax.ShapeDtypeStruct((N, 512), BF16)],
        compiler_params=pltpu.CompilerParams(
            dimension_semantics=("parallel",)),
    )(x, Wc)

    last = NB - 1
    adj_spec = pl.BlockSpec((BLK, N),
                            lambda g, i: (jnp.where(g == 0, i, last), 0))
    res2 = lambda shape: pl.BlockSpec(shape, lambda g, i: (0, 0))
    p0b = lambda w: pl.BlockSpec((BLK, w),
                                 lambda g, i: (jnp.where(g == 0, i, last), 0))
    p1b = lambda w: pl.BlockSpec((BLK, w),
                                 lambda g, i: (jnp.where(g == 1, i, 0), 0))
    p1t = pl.BlockSpec((128, BLK),
                       lambda g, i: (0, jnp.where(g == 1, i, 0)))
    arb2 = pltpu.CompilerParams(
        dimension_semantics=("arbitrary", "arbitrary"),
        vmem_limit_bytes=100 * 1024 * 1024)

    babf = jnp.concatenate([b1, bg1b]).reshape(1, 512)
    fh1, h2, h2t = pl.pallas_call(
        _kf_kernel,
        grid=(2, NB),
        in_specs=[adj_spec, res2((N, 512)), res2((1, 512)),
                  res2((256, 128)), res2((1, 128))],
        out_specs=[p0b(256), p1b(128), p1t],
        out_shape=[jax.ShapeDtypeStruct((N, 256), BF16),
                   jax.ShapeDtypeStruct((N, 128), F32),
                   jax.ShapeDtypeStruct((128, N), BF16)],
        scratch_shapes=[pltpu.VMEM((N, N), BF16),
                        pltpu.VMEM((N, 128), BF16)],
        compiler_params=arb2,
    )(fadj, XWf, babf, Wg2b.astype(BF16), bg2b.reshape(1, 128))

    babs = jnp.concatenate([b1, bg1a]).reshape(1, 512)
    w23 = jnp.concatenate([W2, W3], axis=1).astype(BF16)
    bc = jnp.concatenate([b2, b3, b2, b3, bg2a]).reshape(1, 640)
    lastk = NBK - 1
    adjk_spec = pl.BlockSpec((BLKK, N),
                             lambda g, i: (jnp.where(g == 0, i, lastk), 0))
    pk1b = lambda w: pl.BlockSpec((BLKK, w),
                                  lambda g, i: (jnp.where(g == 1, i, 0), 0))
    pk1t = pl.BlockSpec((128, BLKK),
                        lambda g, i: (0, jnp.where(g == 1, i, 0)))
    smu, slv, fmu, flv, h1, h1t = pl.pallas_call(
        _ks_kernel,
        grid=(2, NBK),
        in_specs=[adjk_spec, res2((N, 512)), res2((1, 512)), res2((N, 256)),
                  res2((256, 256)), res2((256, 128)), res2((1, 640))],
        out_specs=[pk1b(128), pk1b(128), pk1b(128), pk1b(128), pk1b(128),
                   pk1t],
        out_shape=[jax.ShapeDtypeStruct((N, 128), F32)] * 5 +
                  [jax.ShapeDtypeStruct((128, N), BF16)],
        scratch_shapes=[pltpu.VMEM((N, N), BF16),
                        pltpu.VMEM((N, 640), BF16)],
        compiler_params=arb2,
    )(sadj, XWs, babs, fh1, w23, Wg2a.astype(BF16), bc)

    he = pl.BlockSpec((BLK, 128), lambda i: (i, 0))
    res = lambda shape: pl.BlockSpec(shape, lambda i: (0, 0))
    A1, A2, emb1, emb2, out, els = pl.pallas_call(
        _ke_kernel,
        grid=(NB,),
        in_specs=[he, he, res((128, N)), res((128, N)), res((256, 256)),
                  res((256, 128)), res((1, 128)), res((128, 16)),
                  res((1, 16)), res((1, 1))],
        out_specs=[pl.BlockSpec((BLK, N), lambda i: (i, 0)),
                   pl.BlockSpec((BLK, N), lambda i: (i, 0)),
                   he, he,
                   pl.BlockSpec((BLK, 16), lambda i: (i, 0)),
                   pl.BlockSpec((1, 1), lambda i: (0, 0))],
        out_shape=[jax.ShapeDtypeStruct((N, N), F32),
                   jax.ShapeDtypeStruct((N, N), F32),
                   jax.ShapeDtypeStruct((N, 128), F32),
                   jax.ShapeDtypeStruct((N, 128), F32),
                   jax.ShapeDtypeStruct((N, 16), F32),
                   jax.ShapeDtypeStruct((1, 1), F32)],
        compiler_params=pltpu.CompilerParams(
            dimension_semantics=("parallel",),
            vmem_limit_bytes=100 * 1024 * 1024),
    )(h1, h2, h1t, h2t, M1, M2, bM2.reshape(1, 128), M3,
      bM3.reshape(1, 16), logit_scale.reshape(1, 1))

    return (out, A1, A2, emb1, emb2, els.reshape(()), smu, slv, fmu, flv)


# speedup vs baseline: 1.1606x; 1.0431x over previous
import jax
import jax.numpy as jnp
from jax.experimental import pallas as pl
from jax.experimental.pallas import tpu as pltpu

N = 4096
F32 = jnp.float32
BF16 = jnp.bfloat16
BLK = 512
NB = N // BLK
BLKK = 256
NBK = N // BLKK
BLK_E = 512
NB_E = N // BLK_E
BLK_S = 512


def _dot(a, b):
    return jnp.dot(a.astype(BF16), b.astype(BF16),
                   preferred_element_type=F32)


def _xw_kernel(x_ref, w1_ref, wg1a_ref, wg1b_ref, os_ref, of_ref):
    x = x_ref[...]
    xw1 = _dot(x, w1_ref[...]).astype(BF16)
    os_ref[:, :256] = xw1
    os_ref[:, 256:512] = _dot(x, wg1a_ref[...]).astype(BF16)
    of_ref[:, :256] = xw1
    of_ref[:, 256:512] = _dot(x, wg1b_ref[...]).astype(BF16)


def _kf_kernel(f_ref, xwf_ref, b1_ref, bg1b_ref, wg2b_ref, bg_ref,
               fh1_ref, h2_ref, h2t_ref, e2_ref,
               fadj_bf, r2s):
    g = pl.program_id(0)
    i = pl.program_id(1)
    rows = pl.ds(i * BLK, BLK)

    @pl.when(g == 0)
    def _phase0():
        fb = f_ref[...].astype(BF16)
        fadj_bf[rows, :] = fb
        xwf = xwf_ref[...]
        fh1 = jax.nn.relu(_dot(fb, xwf[:, :256]) + b1_ref[...])
        t2 = jax.nn.relu(_dot(fb, xwf[:, 256:512]) + bg1b_ref[...])
        fh1_ref[...] = fh1.astype(BF16)
        r2s[rows, :] = _dot(t2.astype(BF16), wg2b_ref[...]).astype(BF16)

    @pl.when(g == 1)
    def _phase1():
        fb = fadj_bf[rows, :]
        h2b = jax.nn.relu(_dot(fb, r2s[...]) + bg_ref[...])
        h2_ref[...] = h2b.astype(BF16)
        h2t_ref[...] = h2b.T.astype(BF16)
        n2 = jnp.sqrt(jnp.sum(h2b * h2b, axis=1, keepdims=True))
        e2_ref[...] = h2b / n2


def _ks_kernel(s_ref, xws_ref, b1_ref, bg1a_ref, fh1_ref, w2_ref, w3_ref,
               wg2a_ref, b2_ref, b3_ref, bg2a_ref, h2_ref, h2t_ref,
               smu_ref, slv_ref, fmu_ref, flv_ref, h1_ref, h1t_ref, a2_ref,
               sadj_bf, r1s):
    g = pl.program_id(0)
    i = pl.program_id(1)
    rows = pl.ds(i * BLKK, BLKK)
    s = g * NBK + i

    @pl.when(g == 0)
    def _phase0():
        sb = s_ref[...].astype(BF16)
        sadj_bf[rows, :] = sb
        xws = xws_ref[...]
        pa = _dot(sb, xws)
        sh1 = jax.nn.relu(pa[:, :256] + b1_ref[...]).astype(BF16)
        t1 = jax.nn.relu(pa[:, 256:512] + bg1a_ref[...]).astype(BF16)
        fh1 = fh1_ref[...]
        r1s[rows, 0:128] = _dot(sh1, w2_ref[...]).astype(BF16)
        r1s[rows, 128:256] = _dot(sh1, w3_ref[...]).astype(BF16)
        r1s[rows, 256:384] = _dot(fh1, w2_ref[...]).astype(BF16)
        r1s[rows, 384:512] = _dot(fh1, w3_ref[...]).astype(BF16)
        r1s[rows, 512:640] = _dot(t1, wg2a_ref[...]).astype(BF16)

    @pl.when(g == 1)
    def _phase1():
        sb = sadj_bf[rows, :]
        p = _dot(sb, r1s[...])
        smu_ref[...] = jax.nn.relu(p[:, 0:128] + b2_ref[...])
        slv_ref[...] = jax.nn.relu(p[:, 128:256] + b3_ref[...])
        fmu_ref[...] = jax.nn.relu(p[:, 256:384] + b2_ref[...])
        flv_ref[...] = jax.nn.relu(p[:, 384:512] + b3_ref[...])
        h1b = jax.nn.relu(p[:, 512:640] + bg2a_ref[...])
        h1_ref[...] = h1b
        h1t_ref[...] = h1b.T.astype(BF16)

    h2rows = h2_ref[pl.ds(s * 128, 128), :]
    a2_ref[...] = jax.nn.sigmoid(_dot(h2rows, h2t_ref[...]))


def _ke_kernel(h1_ref, h2_ref, h1t_ref, m1_ref, m2_ref, bm2_ref,
               m3_ref, bm3_ref, ls_ref,
               a1_ref, e1_ref, out_ref, els_ref):
    r1 = h1_ref[...]
    a1_ref[...] = jax.nn.sigmoid(_dot(r1, h1t_ref[...]))
    n1 = jnp.sqrt(jnp.sum(r1 * r1, axis=1, keepdims=True))
    e1_ref[...] = r1 / n1
    z = jnp.concatenate([r1.astype(BF16), h2_ref[...]], axis=1)
    t = _dot(z, m1_ref[...])
    t = _dot(t, m2_ref[...]) + bm2_ref[...]
    t = _dot(t, m3_ref[...]) + bm3_ref[...]
    m = jnp.max(t, axis=1, keepdims=True)
    out_ref[...] = t - m - jnp.log(jnp.sum(jnp.exp(t - m), axis=1,
                                           keepdims=True))
    els_ref[...] = jnp.exp(ls_ref[...])


def kernel(x, sadj, fadj, W1, b1, W2, b2, W3, b3, Wg1a, bg1a, Wg2a, bg2a,
           Wg1b, bg1b, Wg2b, bg2b, M1, M2, bM2, M3, bM3, logit_scale):
    XWs, XWf = pl.pallas_call(
        _xw_kernel,
        grid=(N // BLK_S,),
        in_specs=[
            pl.BlockSpec((BLK_S, 512), lambda i: (i, 0)),
            pl.BlockSpec((512, 256), lambda i: (0, 0)),
            pl.BlockSpec((512, 256), lambda i: (0, 0)),
            pl.BlockSpec((512, 256), lambda i: (0, 0)),
        ],
        out_specs=[pl.BlockSpec((BLK_S, 512), lambda i: (i, 0)),
                   pl.BlockSpec((BLK_S, 512), lambda i: (i, 0))],
        out_shape=[jax.ShapeDtypeStruct((N, 512), BF16),
                   jax.ShapeDtypeStruct((N, 512), BF16)],
        compiler_params=pltpu.CompilerParams(
            dimension_semantics=("parallel",)),
    )(x, W1, Wg1a, Wg1b)

    last = NB - 1
    adj_spec = pl.BlockSpec((BLK, N),
                            lambda g, i: (jnp.where(g == 0, i, last), 0))
    res2 = lambda shape: pl.BlockSpec(shape, lambda g, i: (0, 0))
    p0b = lambda w: pl.BlockSpec((BLK, w),
                                 lambda g, i: (jnp.where(g == 0, i, last), 0))
    p1b = lambda w: pl.BlockSpec((BLK, w),
                                 lambda g, i: (jnp.where(g == 1, i, 0), 0))
    p1t = pl.BlockSpec((128, BLK),
                       lambda g, i: (0, jnp.where(g == 1, i, 0)))
    arb2 = pltpu.CompilerParams(
        dimension_semantics=("arbitrary", "arbitrary"),
        vmem_limit_bytes=100 * 1024 * 1024)

    b1r = b1.reshape(1, 256)
    fh1, h2, h2t, emb2 = pl.pallas_call(
        _kf_kernel,
        grid=(2, NB),
        in_specs=[adj_spec, res2((N, 512)), res2((1, 256)), res2((1, 256)),
                  res2((256, 128)), res2((1, 128))],
        out_specs=[p0b(256), p1b(128), p1t, p1b(128)],
        out_shape=[jax.ShapeDtypeStruct((N, 256), BF16),
                   jax.ShapeDtypeStruct((N, 128), BF16),
                   jax.ShapeDtypeStruct((128, N), BF16),
                   jax.ShapeDtypeStruct((N, 128), F32)],
        scratch_shapes=[pltpu.VMEM((N, N), BF16),
                        pltpu.VMEM((N, 128), BF16)],
        compiler_params=arb2,
    )(fadj, XWf, b1r, bg1b.reshape(1, 256), Wg2b, bg2b.reshape(1, 128))

    lastk = NBK - 1
    adjk_spec = pl.BlockSpec((BLKK, N),
                             lambda g, i: (jnp.where(g == 0, i, lastk), 0))
    pk0b = lambda w: pl.BlockSpec((BLKK, w),
                                  lambda g, i: (jnp.where(g == 0, i, lastk),
                                                0))
    pk1b = lambda w: pl.BlockSpec((BLKK, w),
                                  lambda g, i: (jnp.where(g == 1, i, 0), 0))
    pk1t = pl.BlockSpec((128, BLKK),
                        lambda g, i: (0, jnp.where(g == 1, i, 0)))
    a2_spec = pl.BlockSpec((128, N), lambda g, i: (g * NBK + i, 0))
    smu, slv, fmu, flv, h1, h1t, A2 = pl.pallas_call(
        _ks_kernel,
        grid=(2, NBK),
        in_specs=[adjk_spec, res2((N, 512)), res2((1, 256)), res2((1, 256)),
                  pk0b(256), res2((256, 128)), res2((256, 128)),
                  res2((256, 128)), res2((1, 128)), res2((1, 128)),
                  res2((1, 128)), res2((N, 128)), res2((128, N))],
        out_specs=[pk1b(128), pk1b(128), pk1b(128), pk1b(128), pk1b(128),
                   pk1t, a2_spec],
        out_shape=[jax.ShapeDtypeStruct((N, 128), F32)] * 5 +
                  [jax.ShapeDtypeStruct((128, N), BF16),
                   jax.ShapeDtypeStruct((N, N), F32)],
        scratch_shapes=[pltpu.VMEM((N, N), BF16),
                        pltpu.VMEM((N, 640), BF16)],
        compiler_params=arb2,
    )(sadj, XWs, b1r, bg1a.reshape(1, 256), fh1, W2, W3, Wg2a,
      b2.reshape(1, 128), b3.reshape(1, 128), bg2a.reshape(1, 128),
      h2, h2t)

    he = pl.BlockSpec((BLK_E, 128), lambda i: (i, 0))
    res = lambda shape: pl.BlockSpec(shape, lambda i: (0, 0))
    A1, emb1, out, els = pl.pallas_call(
        _ke_kernel,
        grid=(NB_E,),
        in_specs=[he, he, res((128, N)), res((256, 256)),
                  res((256, 128)), res((1, 128)), res((128, 16)),
                  res((1, 16)), res((1, 1))],
        out_specs=[pl.BlockSpec((BLK_E, N), lambda i: (i, 0)),
                   he,
                   pl.BlockSpec((BLK_E, 16), lambda i: (i, 0)),
                   pl.BlockSpec((1, 1), lambda i: (0, 0))],
        out_shape=[jax.ShapeDtypeStruct((N, N), F32),
                   jax.ShapeDtypeStruct((N, 128), F32),
                   jax.ShapeDtypeStruct((N, 16), F32),
                   jax.ShapeDtypeStruct((1, 1), F32)],
        compiler_params=pltpu.CompilerParams(
            dimension_semantics=("parallel",),
            vmem_limit_bytes=100 * 1024 * 1024),
    )(h1, h2, h1t, M1, M2, bM2.reshape(1, 128), M3,
      bM3.reshape(1, 16), logit_scale.reshape(1, 1))

    return (out, A1, A2, emb1, emb2, els.reshape(()), smu, slv, fmu, flv)


# speedup vs baseline: 1.2002x; 1.0341x over previous
import jax
import jax.numpy as jnp
from jax.experimental import pallas as pl
from jax.experimental.pallas import tpu as pltpu

N = 4096
F32 = jnp.float32
BF16 = jnp.bfloat16
BLK = 512
NB = N // BLK
BLKK = 256
NBK = N // BLKK
BLK_E = 512
NB_E = N // BLK_E
BLK_S = 1024


def _dot(a, b):
    return jnp.dot(a.astype(BF16), b.astype(BF16),
                   preferred_element_type=F32)


def _xw_kernel(x_ref, w1_ref, wg1a_ref, wg1b_ref, os_ref, of_ref):
    x = x_ref[...]
    xw1 = _dot(x, w1_ref[...]).astype(BF16)
    os_ref[:, :256] = xw1
    os_ref[:, 256:512] = _dot(x, wg1a_ref[...]).astype(BF16)
    of_ref[:, :256] = xw1
    of_ref[:, 256:512] = _dot(x, wg1b_ref[...]).astype(BF16)


def _kf_kernel(f_ref, xwf_ref, b1_ref, bg1b_ref, wg2b_ref, bg_ref,
               fh1_ref, h2_ref, h2t_ref, e2_ref,
               fadj_bf, r2s):
    g = pl.program_id(0)
    i = pl.program_id(1)
    rows = pl.ds(i * BLK, BLK)

    @pl.when(g == 0)
    def _phase0():
        fb = f_ref[...].astype(BF16)
        fadj_bf[rows, :] = fb
        xwf = xwf_ref[...]
        fh1 = jax.nn.relu(_dot(fb, xwf[:, :256]) + b1_ref[...])
        t2 = jax.nn.relu(_dot(fb, xwf[:, 256:512]) + bg1b_ref[...])
        fh1_ref[...] = fh1.astype(BF16)
        r2s[rows, :] = _dot(t2.astype(BF16), wg2b_ref[...]).astype(BF16)

    @pl.when(g == 1)
    def _phase1():
        fb = fadj_bf[rows, :]
        h2b = jax.nn.relu(_dot(fb, r2s[...]) + bg_ref[...])
        h2_ref[...] = h2b.astype(BF16)
        h2t_ref[...] = h2b.T.astype(BF16)
        n2 = jnp.sqrt(jnp.sum(h2b * h2b, axis=1, keepdims=True))
        e2_ref[...] = h2b / n2


def _ks_kernel(s_ref, xws_ref, b1_ref, bg1a_ref, fh1_ref, w2_ref, w3_ref,
               wg2a_ref, b2_ref, b3_ref, bg2a_ref,
               smu_ref, slv_ref, fmu_ref, flv_ref, h1_ref, h1t_ref,
               sadj_bf, r1s):
    g = pl.program_id(0)
    i = pl.program_id(1)
    rows = pl.ds(i * BLKK, BLKK)

    @pl.when(g == 0)
    def _phase0():
        sb = s_ref[...].astype(BF16)
        sadj_bf[rows, :] = sb
        xws = xws_ref[...]
        pa = _dot(sb, xws)
        sh1 = jax.nn.relu(pa[:, :256] + b1_ref[...]).astype(BF16)
        t1 = jax.nn.relu(pa[:, 256:512] + bg1a_ref[...]).astype(BF16)
        fh1 = fh1_ref[...]
        r1s[rows, 0:128] = _dot(sh1, w2_ref[...]).astype(BF16)
        r1s[rows, 128:256] = _dot(sh1, w3_ref[...]).astype(BF16)
        r1s[rows, 256:384] = _dot(fh1, w2_ref[...]).astype(BF16)
        r1s[rows, 384:512] = _dot(fh1, w3_ref[...]).astype(BF16)
        r1s[rows, 512:640] = _dot(t1, wg2a_ref[...]).astype(BF16)

    @pl.when(g == 1)
    def _phase1():
        sb = sadj_bf[rows, :]
        p = _dot(sb, r1s[...])
        smu_ref[...] = jax.nn.relu(p[:, 0:128] + b2_ref[...])
        slv_ref[...] = jax.nn.relu(p[:, 128:256] + b3_ref[...])
        fmu_ref[...] = jax.nn.relu(p[:, 256:384] + b2_ref[...])
        flv_ref[...] = jax.nn.relu(p[:, 384:512] + b3_ref[...])
        h1b = jax.nn.relu(p[:, 512:640] + bg2a_ref[...])
        h1_ref[...] = h1b
        h1t_ref[...] = h1b.T.astype(BF16)


def _ke_kernel(h1_ref, h2_ref, h1t_ref, h2t_ref, m1_ref, m2_ref, bm2_ref,
               m3_ref, bm3_ref, ls_ref,
               a1_ref, a2_ref, e1_ref, out_ref, els_ref):
    r1 = h1_ref[...]
    a1_ref[...] = jax.nn.sigmoid(_dot(r1, h1t_ref[...]))
    a2_ref[...] = jax.nn.sigmoid(_dot(h2_ref[...], h2t_ref[...]))
    n1 = jnp.sqrt(jnp.sum(r1 * r1, axis=1, keepdims=True))
    e1_ref[...] = r1 / n1
    z = jnp.concatenate([r1.astype(BF16), h2_ref[...]], axis=1)
    t = _dot(z, m1_ref[...])
    t = _dot(t, m2_ref[...]) + bm2_ref[...]
    t = _dot(t, m3_ref[...]) + bm3_ref[...]
    m = jnp.max(t, axis=1, keepdims=True)
    out_ref[...] = t - m - jnp.log(jnp.sum(jnp.exp(t - m), axis=1,
                                           keepdims=True))
    els_ref[...] = jnp.exp(ls_ref[...])


def kernel(x, sadj, fadj, W1, b1, W2, b2, W3, b3, Wg1a, bg1a, Wg2a, bg2a,
           Wg1b, bg1b, Wg2b, bg2b, M1, M2, bM2, M3, bM3, logit_scale):
    XWs, XWf = pl.pallas_call(
        _xw_kernel,
        grid=(N // BLK_S,),
        in_specs=[
            pl.BlockSpec((BLK_S, 512), lambda i: (i, 0)),
            pl.BlockSpec((512, 256), lambda i: (0, 0)),
            pl.BlockSpec((512, 256), lambda i: (0, 0)),
            pl.BlockSpec((512, 256), lambda i: (0, 0)),
        ],
        out_specs=[pl.BlockSpec((BLK_S, 512), lambda i: (i, 0)),
                   pl.BlockSpec((BLK_S, 512), lambda i: (i, 0))],
        out_shape=[jax.ShapeDtypeStruct((N, 512), BF16),
                   jax.ShapeDtypeStruct((N, 512), BF16)],
        compiler_params=pltpu.CompilerParams(
            dimension_semantics=("parallel",)),
    )(x, W1, Wg1a, Wg1b)

    last = NB - 1
    adj_spec = pl.BlockSpec((BLK, N),
                            lambda g, i: (jnp.where(g == 0, i, last), 0))
    res2 = lambda shape: pl.BlockSpec(shape, lambda g, i: (0, 0))
    p0b = lambda w: pl.BlockSpec((BLK, w),
                                 lambda g, i: (jnp.where(g == 0, i, last), 0))
    p1b = lambda w: pl.BlockSpec((BLK, w),
                                 lambda g, i: (jnp.where(g == 1, i, 0), 0))
    p1t = pl.BlockSpec((128, BLK),
                       lambda g, i: (0, jnp.where(g == 1, i, 0)))
    arb2 = pltpu.CompilerParams(
        dimension_semantics=("arbitrary", "arbitrary"),
        vmem_limit_bytes=100 * 1024 * 1024)

    b1r = b1.reshape(1, 256)
    fh1, h2, h2t, emb2 = pl.pallas_call(
        _kf_kernel,
        grid=(2, NB),
        in_specs=[adj_spec, res2((N, 512)), res2((1, 256)), res2((1, 256)),
                  res2((256, 128)), res2((1, 128))],
        out_specs=[p0b(256), p1b(128), p1t, p1b(128)],
        out_shape=[jax.ShapeDtypeStruct((N, 256), BF16),
                   jax.ShapeDtypeStruct((N, 128), BF16),
                   jax.ShapeDtypeStruct((128, N), BF16),
                   jax.ShapeDtypeStruct((N, 128), F32)],
        scratch_shapes=[pltpu.VMEM((N, N), BF16),
                        pltpu.VMEM((N, 128), BF16)],
        compiler_params=arb2,
    )(fadj, XWf, b1r, bg1b.reshape(1, 256), Wg2b, bg2b.reshape(1, 128))

    lastk = NBK - 1
    adjk_spec = pl.BlockSpec((BLKK, N),
                             lambda g, i: (jnp.where(g == 0, i, lastk), 0))
    pk0b = lambda w: pl.BlockSpec((BLKK, w),
                                  lambda g, i: (jnp.where(g == 0, i, lastk),
                                                0))
    pk1b = lambda w: pl.BlockSpec((BLKK, w),
                                  lambda g, i: (jnp.where(g == 1, i, 0), 0))
    pk1t = pl.BlockSpec((128, BLKK),
                        lambda g, i: (0, jnp.where(g == 1, i, 0)))
    smu, slv, fmu, flv, h1, h1t = pl.pallas_call(
        _ks_kernel,
        grid=(2, NBK),
        in_specs=[adjk_spec, res2((N, 512)), res2((1, 256)), res2((1, 256)),
                  pk0b(256), res2((256, 128)), res2((256, 128)),
                  res2((256, 128)), res2((1, 128)), res2((1, 128)),
                  res2((1, 128))],
        out_specs=[pk1b(128), pk1b(128), pk1b(128), pk1b(128), pk1b(128),
                   pk1t],
        out_shape=[jax.ShapeDtypeStruct((N, 128), F32)] * 5 +
                  [jax.ShapeDtypeStruct((128, N), BF16)],
        scratch_shapes=[pltpu.VMEM((N, N), BF16),
                        pltpu.VMEM((N, 640), BF16)],
        compiler_params=arb2,
    )(sadj, XWs, b1r, bg1a.reshape(1, 256), fh1, W2, W3, Wg2a,
      b2.reshape(1, 128), b3.reshape(1, 128), bg2a.reshape(1, 128))

    he = pl.BlockSpec((BLK_E, 128), lambda i: (i, 0))
    res = lambda shape: pl.BlockSpec(shape, lambda i: (0, 0))
    A1, A2, emb1, out, els = pl.pallas_call(
        _ke_kernel,
        grid=(NB_E,),
        in_specs=[he, he, res((128, N)), res((128, N)), res((256, 256)),
                  res((256, 128)), res((1, 128)), res((128, 16)),
                  res((1, 16)), res((1, 1))],
        out_specs=[pl.BlockSpec((BLK_E, N), lambda i: (i, 0)),
                   pl.BlockSpec((BLK_E, N), lambda i: (i, 0)),
                   he,
                   pl.BlockSpec((BLK_E, 16), lambda i: (i, 0)),
                   pl.BlockSpec((1, 1), lambda i: (0, 0))],
        out_shape=[jax.ShapeDtypeStruct((N, N), F32),
                   jax.ShapeDtypeStruct((N, N), F32),
                   jax.ShapeDtypeStruct((N, 128), F32),
                   jax.ShapeDtypeStruct((N, 16), F32),
                   jax.ShapeDtypeStruct((1, 1), F32)],
        compiler_params=pltpu.CompilerParams(
            dimension_semantics=("parallel",),
            vmem_limit_bytes=100 * 1024 * 1024),
    )(h1, h2, h1t, h2t, M1, M2, bM2.reshape(1, 128), M3,
      bM3.reshape(1, 16), logit_scale.reshape(1, 1))

    return (out, A1, A2, emb1, emb2, els.reshape(()), smu, slv, fmu, flv)


# speedup vs baseline: 1.2072x; 1.0058x over previous
import jax
import jax.numpy as jnp
from jax.experimental import pallas as pl
from jax.experimental.pallas import tpu as pltpu

N = 4096
F32 = jnp.float32
BF16 = jnp.bfloat16
BLK = 512
NB = N // BLK
BLKK = 256
NBK = N // BLKK
BLK_E = 512
NB_E = N // BLK_E
BLK_S = 1024


def _dot(a, b):
    return jnp.dot(a.astype(BF16), b.astype(BF16),
                   preferred_element_type=F32)


def _xw_kernel(x_ref, w1_ref, wg1a_ref, wg1b_ref, os_ref, of_ref):
    x = x_ref[...]
    xw1 = _dot(x, w1_ref[...]).astype(BF16)
    os_ref[:, :256] = xw1
    os_ref[:, 256:512] = _dot(x, wg1a_ref[...]).astype(BF16)
    of_ref[:, :256] = xw1
    of_ref[:, 256:512] = _dot(x, wg1b_ref[...]).astype(BF16)


def _kf_kernel(f_ref, xwf_ref, b1_ref, bg1b_ref, wg2b_ref, bg_ref,
               fh1_ref, h2_ref, h2t_ref, e2_ref,
               fadj_bf, r2s):
    g = pl.program_id(0)
    i = pl.program_id(1)
    rows = pl.ds(i * BLK, BLK)

    @pl.when(g == 0)
    def _phase0():
        fadj_bf[rows, :] = f_ref[...].astype(BF16)
        fb = fadj_bf[rows, :]
        xwf = xwf_ref[...]
        fh1 = jax.nn.relu(_dot(fb, xwf[:, :256]) + b1_ref[...])
        t2 = jax.nn.relu(_dot(fb, xwf[:, 256:512]) + bg1b_ref[...])
        fh1_ref[...] = fh1.astype(BF16)
        r2s[rows, :] = _dot(t2.astype(BF16), wg2b_ref[...]).astype(BF16)

    @pl.when(g == 1)
    def _phase1():
        fb = fadj_bf[rows, :]
        h2b = jax.nn.relu(_dot(fb, r2s[...]) + bg_ref[...])
        h2_ref[...] = h2b.astype(BF16)
        h2t_ref[...] = h2b.T.astype(BF16)
        n2 = jnp.sqrt(jnp.sum(h2b * h2b, axis=1, keepdims=True))
        e2_ref[...] = h2b / n2


def _ks_kernel(s_ref, xws_ref, b1_ref, bg1a_ref, fh1_ref, w2_ref, w3_ref,
               wg2a_ref, b2_ref, b3_ref, bg2a_ref,
               smu_ref, slv_ref, fmu_ref, flv_ref, h1_ref, h1t_ref,
               sadj_bf, r1s):
    g = pl.program_id(0)
    i = pl.program_id(1)
    rows = pl.ds(i * BLKK, BLKK)

    @pl.when(g == 0)
    def _phase0():
        sadj_bf[rows, :] = s_ref[...].astype(BF16)
        sb = sadj_bf[rows, :]
        xws = xws_ref[...]
        pa = _dot(sb, xws)
        sh1 = jax.nn.relu(pa[:, :256] + b1_ref[...]).astype(BF16)
        t1 = jax.nn.relu(pa[:, 256:512] + bg1a_ref[...]).astype(BF16)
        fh1 = fh1_ref[...]
        r1s[rows, 0:128] = _dot(sh1, w2_ref[...]).astype(BF16)
        r1s[rows, 128:256] = _dot(sh1, w3_ref[...]).astype(BF16)
        r1s[rows, 256:384] = _dot(fh1, w2_ref[...]).astype(BF16)
        r1s[rows, 384:512] = _dot(fh1, w3_ref[...]).astype(BF16)
        r1s[rows, 512:640] = _dot(t1, wg2a_ref[...]).astype(BF16)

    @pl.when(g == 1)
    def _phase1():
        sb = sadj_bf[rows, :]
        p = _dot(sb, r1s[...])
        smu_ref[...] = jax.nn.relu(p[:, 0:128] + b2_ref[...])
        slv_ref[...] = jax.nn.relu(p[:, 128:256] + b3_ref[...])
        fmu_ref[...] = jax.nn.relu(p[:, 256:384] + b2_ref[...])
        flv_ref[...] = jax.nn.relu(p[:, 384:512] + b3_ref[...])
        h1b = jax.nn.relu(p[:, 512:640] + bg2a_ref[...])
        h1_ref[...] = h1b
        h1t_ref[...] = h1b.T.astype(BF16)


def _ke_kernel(h1_ref, h2_ref, h1t_ref, h2t_ref, m1_ref, m2_ref, bm2_ref,
               m3_ref, bm3_ref, ls_ref,
               a1_ref, a2_ref, e1_ref, out_ref, els_ref):
    r1 = h1_ref[...]
    a1_ref[...] = jax.nn.sigmoid(_dot(r1, h1t_ref[...]))
    a2_ref[...] = jax.nn.sigmoid(_dot(h2_ref[...], h2t_ref[...]))
    n1 = jnp.sqrt(jnp.sum(r1 * r1, axis=1, keepdims=True))
    e1_ref[...] = r1 / n1
    z = jnp.concatenate([r1.astype(BF16), h2_ref[...]], axis=1)
    t = _dot(z, m1_ref[...])
    t = _dot(t, m2_ref[...]) + bm2_ref[...]
    t = _dot(t, m3_ref[...]) + bm3_ref[...]
    m = jnp.max(t, axis=1, keepdims=True)
    out_ref[...] = t - m - jnp.log(jnp.sum(jnp.exp(t - m), axis=1,
                                           keepdims=True))
    els_ref[...] = jnp.exp(ls_ref[...])


def kernel(x, sadj, fadj, W1, b1, W2, b2, W3, b3, Wg1a, bg1a, Wg2a, bg2a,
           Wg1b, bg1b, Wg2b, bg2b, M1, M2, bM2, M3, bM3, logit_scale):
    XWs, XWf = pl.pallas_call(
        _xw_kernel,
        grid=(N // BLK_S,),
        in_specs=[
            pl.BlockSpec((BLK_S, 512), lambda i: (i, 0)),
            pl.BlockSpec((512, 256), lambda i: (0, 0)),
            pl.BlockSpec((512, 256), lambda i: (0, 0)),
            pl.BlockSpec((512, 256), lambda i: (0, 0)),
        ],
        out_specs=[pl.BlockSpec((BLK_S, 512), lambda i: (i, 0)),
                   pl.BlockSpec((BLK_S, 512), lambda i: (i, 0))],
        out_shape=[jax.ShapeDtypeStruct((N, 512), BF16),
                   jax.ShapeDtypeStruct((N, 512), BF16)],
        compiler_params=pltpu.CompilerParams(
            dimension_semantics=("parallel",)),
    )(x, W1, Wg1a, Wg1b)

    last = NB - 1
    adj_spec = pl.BlockSpec((BLK, N),
                            lambda g, i: (jnp.where(g == 0, i, last), 0))
    res2 = lambda shape: pl.BlockSpec(shape, lambda g, i: (0, 0))
    p0b = lambda w: pl.BlockSpec((BLK, w),
                                 lambda g, i: (jnp.where(g == 0, i, last), 0))
    p1b = lambda w: pl.BlockSpec((BLK, w),
                                 lambda g, i: (jnp.where(g == 1, i, 0), 0))
    p1t = pl.BlockSpec((128, BLK),
                       lambda g, i: (0, jnp.where(g == 1, i, 0)))
    arb2 = pltpu.CompilerParams(
        dimension_semantics=("arbitrary", "arbitrary"),
        vmem_limit_bytes=100 * 1024 * 1024)

    b1r = b1.reshape(1, 256)
    fh1, h2, h2t, emb2 = pl.pallas_call(
        _kf_kernel,
        grid=(2, NB),
        in_specs=[adj_spec, res2((N, 512)), res2((1, 256)), res2((1, 256)),
                  res2((256, 128)), res2((1, 128))],
        out_specs=[p0b(256), p1b(128), p1t, p1b(128)],
        out_shape=[jax.ShapeDtypeStruct((N, 256), BF16),
                   jax.ShapeDtypeStruct((N, 128), BF16),
                   jax.ShapeDtypeStruct((128, N), BF16),
                   jax.ShapeDtypeStruct((N, 128), F32)],
        scratch_shapes=[pltpu.VMEM((N, N), BF16),
                        pltpu.VMEM((N, 128), BF16)],
        compiler_params=arb2,
    )(fadj, XWf, b1r, bg1b.reshape(1, 256), Wg2b, bg2b.reshape(1, 128))

    lastk = NBK - 1
    adjk_spec = pl.BlockSpec((BLKK, N),
                             lambda g, i: (jnp.where(g == 0, i, lastk), 0))
    pk0b = lambda w: pl.BlockSpec((BLKK, w),
                                  lambda g, i: (jnp.where(g == 0, i, lastk),
                                                0))
    pk1b = lambda w: pl.BlockSpec((BLKK, w),
                                  lambda g, i: (jnp.where(g == 1, i, 0), 0))
    pk1t = pl.BlockSpec((128, BLKK),
                        lambda g, i: (0, jnp.where(g == 1, i, 0)))
    smu, slv, fmu, flv, h1, h1t = pl.pallas_call(
        _ks_kernel,
        grid=(2, NBK),
        in_specs=[adjk_spec, res2((N, 512)), res2((1, 256)), res2((1, 256)),
                  pk0b(256), res2((256, 128)), res2((256, 128)),
                  res2((256, 128)), res2((1, 128)), res2((1, 128)),
                  res2((1, 128))],
        out_specs=[pk1b(128), pk1b(128), pk1b(128), pk1b(128), pk1b(128),
                   pk1t],
        out_shape=[jax.ShapeDtypeStruct((N, 128), F32)] * 5 +
                  [jax.ShapeDtypeStruct((128, N), BF16)],
        scratch_shapes=[pltpu.VMEM((N, N), BF16),
                        pltpu.VMEM((N, 640), BF16)],
        compiler_params=arb2,
    )(sadj, XWs, b1r, bg1a.reshape(1, 256), fh1, W2, W3, Wg2a,
      b2.reshape(1, 128), b3.reshape(1, 128), bg2a.reshape(1, 128))

    he = pl.BlockSpec((BLK_E, 128), lambda i: (i, 0))
    res = lambda shape: pl.BlockSpec(shape, lambda i: (0, 0))
    A1, A2, emb1, out, els = pl.pallas_call(
        _ke_kernel,
        grid=(NB_E,),
        in_specs=[he, he, res((128, N)), res((128, N)), res((256, 256)),
                  res((256, 128)), res((1, 128)), res((128, 16)),
                  res((1, 16)), res((1, 1))],
        out_specs=[pl.BlockSpec((BLK_E, N), lambda i: (i, 0)),
                   pl.BlockSpec((BLK_E, N), lambda i: (i, 0)),
                   he,
                   pl.BlockSpec((BLK_E, 16), lambda i: (i, 0)),
                   pl.BlockSpec((1, 1), lambda i: (0, 0))],
        out_shape=[jax.ShapeDtypeStruct((N, N), F32),
                   jax.ShapeDtypeStruct((N, N), F32),
                   jax.ShapeDtypeStruct((N, 128), F32),
                   jax.ShapeDtypeStruct((N, 16), F32),
                   jax.ShapeDtypeStruct((1, 1), F32)],
        compiler_params=pltpu.CompilerParams(
            dimension_semantics=("parallel",),
            vmem_limit_bytes=100 * 1024 * 1024),
    )(h1, h2, h1t, h2t, M1, M2, bM2.reshape(1, 128), M3,
      bM3.reshape(1, 16), logit_scale.reshape(1, 1))

    return (out, A1, A2, emb1, emb2, els.reshape(()), smu, slv, fmu, flv)
